# baseline (speedup 1.0000x reference)
import functools

import jax
import jax.numpy as jnp
from jax import lax
from jax.experimental import pallas as pl
from jax.experimental.pallas import tpu as pltpu
from jax.experimental.pallas import tpu_sc as plsc

V, E, NCLS, B, S = 1000, 64, 10, 4096, 200
NC, NS, L = 2, 16, 16
NW = NC * NS
BPW = B // NW
NFULL = S // L
NREM = S - NFULL * L
EW = E // 2
CPAD = 128

_mesh = plsc.VectorSubcoreMesh(
    core_axis_name="c", subcore_axis_name="s", num_cores=2)


def _bcast_lane(vec, j):
    return lax.gather(
        vec,
        jnp.full((L, 1), j, jnp.int32),
        lax.GatherDimensionNumbers(
            offset_dims=(), collapsed_slice_dims=(0,), start_index_map=(0,)),
        (1,),
        mode=lax.GatherScatterMode.PROMISE_IN_BOUNDS,
    )


@functools.partial(
    pl.kernel,
    out_type=jax.ShapeDtypeStruct((B * EW,), jnp.int32),
    mesh=_mesh,
    scratch_types=[
        pltpu.VMEM((BPW * S + L - NREM,), jnp.int32),
        pltpu.VMEM((V * EW,), jnp.int32),
        pltpu.VMEM((BPW * EW,), jnp.int32),
    ],
    compiler_params=pltpu.CompilerParams(needs_layout_passes=False),
)
def _pool_sc(idx_hbm, tbl_hbm, out_hbm, idx_v, tbl_v, out_v):
    wid = lax.axis_index("s") * NC + lax.axis_index("c")
    base = wid * BPW
    pltpu.sync_copy(tbl_hbm, tbl_v)
    pltpu.sync_copy(idx_hbm.at[pl.ds(base * S, BPW * S)],
                    idx_v.at[pl.ds(0, BPW * S)])
    lanes = lax.iota(jnp.int32, L)
    ninf = jnp.full((2 * L,), -jnp.inf, jnp.bfloat16)

    def gather_max(addr, acc):
        row = plsc.bitcast(plsc.load_gather(tbl_v, [addr]), jnp.bfloat16)
        return jnp.maximum(acc, row)

    def row_body(row, _):
        def chunk_body(c, accs):
            idxv = idx_v[pl.ds(row * S + c * L, L)]
            a, b2 = accs
            for j in range(L):
                addr = _bcast_lane(idxv, j) * EW + lanes
                a = gather_max(addr, a)
                b2 = gather_max(addr + L, b2)
            return (a, b2)

        accs = lax.fori_loop(0, NFULL, chunk_body, (ninf, ninf))
        idxv = idx_v[pl.ds(row * S + NFULL * L, L)]
        a, b2 = accs
        for j in range(NREM):
            addr = _bcast_lane(idxv, j) * EW + lanes
            a = gather_max(addr, a)
            b2 = gather_max(addr + L, b2)
        out_v[pl.ds(row * EW, L)] = plsc.bitcast(a, jnp.int32)
        out_v[pl.ds(row * EW + L, L)] = plsc.bitcast(b2, jnp.int32)
        return 0

    lax.fori_loop(0, BPW, row_body, 0)
    pltpu.sync_copy(out_v, out_hbm.at[pl.ds(base * EW, BPW * EW)])


BLK = 512


def _dense_tc(x_ref, w_ref, b_ref, o_ref):
    logits = jnp.dot(x_ref[...], w_ref[...],
                     preferred_element_type=jnp.float32) + b_ref[...]
    m = jnp.max(logits, axis=1, keepdims=True)
    e = jnp.exp(logits - m)
    o_ref[...] = e / jnp.sum(e, axis=1, keepdims=True)


_dense_call = pl.pallas_call(
    _dense_tc,
    grid=(B // BLK,),
    in_specs=[
        pl.BlockSpec((BLK, E), lambda i: (i, 0)),
        pl.BlockSpec((E, CPAD), lambda i: (0, 0)),
        pl.BlockSpec((1, CPAD), lambda i: (0, 0)),
    ],
    out_specs=pl.BlockSpec((BLK, CPAD), lambda i: (i, 0)),
    out_shape=jax.ShapeDtypeStruct((B, CPAD), jnp.float32),
)


def kernel(indices, table, W, b):
    tbl_p = lax.bitcast_convert_type(
        table.astype(jnp.bfloat16).reshape(V, EW, 2), jnp.int32).reshape(-1)
    pooled = _pool_sc(indices.reshape(-1), tbl_p)
    x = lax.bitcast_convert_type(
        pooled.reshape(B, EW), jnp.bfloat16).reshape(B, E)
    w_p = jnp.pad(W.astype(jnp.bfloat16), ((0, 0), (0, CPAD - NCLS)))
    b_p = jnp.concatenate(
        [b, jnp.full((CPAD - NCLS,), -1e30, jnp.float32)]).reshape(1, CPAD)
    return _dense_call(x, w_p, b_p)[:, :NCLS]
